# Initial kernel scaffold; baseline (speedup 1.0000x reference)
import jax
import jax.numpy as jnp
from jax.experimental import pallas as pl


def kernel(x, edge_index, W1, b1, W2, b2):
    raise NotImplementedError("write your pallas kernel here")


# speedup vs baseline: 10.6099x; 10.6099x over previous
import functools

import jax
import jax.numpy as jnp
from jax import lax
from jax.experimental import pallas as pl
from jax.experimental.pallas import tpu as pltpu
from jax.experimental.pallas import tpu_sc as plsc

N = 10000
IN_CH = 256
HID = 512
OUT_CH = 256
CH = 128
E = 160000
K = 128
ROWS = 1280
EPAD = ROWS * K
DUMMY = N
NPAD = 10240
NC, NS = 2, 16
RP = ROWS // NS
HK = RP // 2
RD = ROWS // (NC * NS)
NROWS_T = NPAD // NS
ZROWS_T = NPAD // NS
DEGW = 128
RB = 2000


def _mesh():
    return plsc.VectorSubcoreMesh(
        core_axis_name="c", subcore_axis_name="s",
        num_cores=NC, num_subcores=NS)


@functools.partial(
    pl.kernel,
    out_type=jax.ShapeDtypeStruct((NC, NPAD, DEGW), jnp.float32),
    mesh=_mesh(),
    scratch_types=[
        pltpu.VMEM((RD, K), jnp.int32),
        pltpu.VMEM((K, DEGW), jnp.float32),
        pltpu.VMEM_SHARED((NPAD, DEGW), jnp.float32),
    ],
)
def _deg_kernel(dst_hbm, zeros_hbm, ones_hbm, out_hbm, dst_v, ones_v, acc_sh):
    c = lax.axis_index("c")
    s = lax.axis_index("s")
    w = c * NS + s
    pltpu.sync_copy(zeros_hbm.at[pl.ds(s * ZROWS_T, ZROWS_T)],
                    acc_sh.at[pl.ds(s * ZROWS_T, ZROWS_T)])
    pltpu.sync_copy(dst_hbm.at[pl.ds(w * RD, RD)], dst_v)
    pltpu.sync_copy(ones_hbm, ones_v)
    plsc.subcore_barrier()

    def body(j, carry):
        pltpu.sync_copy(ones_v, acc_sh.at[dst_v.at[j]], add=True)
        return carry

    lax.fori_loop(0, RD, body, 0)
    plsc.subcore_barrier()
    pltpu.sync_copy(acc_sh.at[pl.ds(s * ZROWS_T, ZROWS_T)],
                    out_hbm.at[c, pl.ds(s * ZROWS_T, ZROWS_T)])


@functools.partial(
    pl.kernel,
    out_type=jax.ShapeDtypeStruct((NC, NPAD, CH), jnp.float32),
    mesh=_mesh(),
    scratch_types=[
        pltpu.VMEM((HK, K), jnp.int32),
        pltpu.VMEM((HK, K), jnp.int32),
        pltpu.VMEM((K, CH), jnp.float32),
        pltpu.VMEM((K, CH), jnp.float32),
        pltpu.VMEM_SHARED((NPAD, CH), jnp.float32),
        pltpu.SemaphoreType.DMA,
        pltpu.SemaphoreType.DMA,
    ],
)
def _prop_kernel(ys_hbm, src_hbm, dst_hbm, out_hbm,
                 src_v, dst_v, gb0, gb1, acc_sh, sem0, sem1):
    c = lax.axis_index("c")
    s = lax.axis_index("s")
    pltpu.sync_copy(ys_hbm.at[c, pl.ds(s * NROWS_T, NROWS_T)],
                    acc_sh.at[pl.ds(s * NROWS_T, NROWS_T)])
    plsc.subcore_barrier()

    gbufs = (gb0, gb1)
    sems = (sem0, sem1)

    def start(chunk, b):
        pltpu.async_copy(ys_hbm.at[c].at[src_v.at[chunk]], gbufs[b], sems[b])

    def wait(chunk, b):
        pltpu.make_async_copy(ys_hbm.at[c].at[src_v.at[chunk]],
                              gbufs[b], sems[b]).wait()

    def scat(chunk, b):
        pltpu.sync_copy(gbufs[b], acc_sh.at[dst_v.at[chunk]], add=True)

    def body(g, carry):
        ch0 = g * 2
        for b in range(2):
            wait(ch0 + b, b)
            scat(ch0 + b, b)
            start(ch0 + b + 2, b)
        return carry

    for h in range(2):
        base = s * RP + h * HK
        pltpu.sync_copy(src_hbm.at[pl.ds(base, HK)], src_v)
        pltpu.sync_copy(dst_hbm.at[pl.ds(base, HK)], dst_v)
        start(0, 0)
        start(1, 1)
        lax.fori_loop(0, HK // 2 - 1, body, 0)
        for b in range(2):
            wait(HK - 2 + b, b)
            scat(HK - 2 + b, b)
    plsc.subcore_barrier()
    pltpu.sync_copy(acc_sh.at[pl.ds(s * NROWS_T, NROWS_T)],
                    out_hbm.at[c, pl.ds(s * NROWS_T, NROWS_T)])


def _dis_of(p):
    return lax.rsqrt(1.0 + p[0, :, 0] + p[1, :, 0])


def _prep_body(deg_ref, x_ref, ys_ref):
    dis = _dis_of(deg_ref[...])
    xs = x_ref[...] * dis[:, None]
    ys_ref[0] = xs[:, :CH]
    ys_ref[1] = xs[:, CH:]


def _dense_body(deg_ref, s1_ref, w1_ref, b1_ref, w2_ref, out_ref):
    dis = _dis_of(deg_ref[...])
    s1 = jnp.concatenate([s1_ref[0], s1_ref[1]], axis=1) * dis[:, None]
    h = jnp.dot(s1, w1_ref[...], preferred_element_type=jnp.float32)
    h = jnp.maximum(h + b1_ref[...], 0.0)
    g = jnp.dot(h, w2_ref[...], preferred_element_type=jnp.float32)
    gs = g * dis[:, None]
    out_ref[0] = gs[:, :CH]
    out_ref[1] = gs[:, CH:]


def _out_body(deg_ref, s2_ref, b2_ref, out_ref):
    dis = _dis_of(deg_ref[...])
    p2 = jnp.concatenate([s2_ref[0], s2_ref[1]], axis=1) * dis[:, None]
    p2 = p2 + b2_ref[...]
    sh = p2 - jnp.max(p2, axis=1, keepdims=True)
    lse = jnp.log(jnp.sum(jnp.exp(sh), axis=1, keepdims=True))
    out_ref[...] = sh - lse


_DEG_SPEC = pl.BlockSpec((NC, RB, DEGW), lambda i: (0, i, 0))
_SPLIT_SPEC = pl.BlockSpec((NC, RB, CH), lambda i: (0, i, 0))


def _prep(deg_parts, x):
    return pl.pallas_call(
        _prep_body,
        grid=(N // RB,),
        in_specs=[_DEG_SPEC, pl.BlockSpec((RB, IN_CH), lambda i: (i, 0))],
        out_specs=_SPLIT_SPEC,
        out_shape=jax.ShapeDtypeStruct((NC, NPAD, CH), jnp.float32),
    )(deg_parts, x)


def _dense(deg_parts, s1, W1, b1, W2):
    return pl.pallas_call(
        _dense_body,
        grid=(N // RB,),
        in_specs=[
            _DEG_SPEC,
            _SPLIT_SPEC,
            pl.BlockSpec((IN_CH, HID), lambda i: (0, 0)),
            pl.BlockSpec((1, HID), lambda i: (0, 0)),
            pl.BlockSpec((HID, OUT_CH), lambda i: (0, 0)),
        ],
        out_specs=_SPLIT_SPEC,
        out_shape=jax.ShapeDtypeStruct((NC, NPAD, CH), jnp.float32),
    )(deg_parts, s1, W1, b1, W2)


def _final(deg_parts, s2, b2):
    return pl.pallas_call(
        _out_body,
        grid=(N // RB,),
        in_specs=[
            _DEG_SPEC,
            _SPLIT_SPEC,
            pl.BlockSpec((1, OUT_CH), lambda i: (0, 0)),
        ],
        out_specs=pl.BlockSpec((RB, OUT_CH), lambda i: (i, 0)),
        out_shape=jax.ShapeDtypeStruct((N, OUT_CH), jnp.float32),
    )(deg_parts, s2, b2)


def _prop_jnp(ys, src, dst):
    return ys.at[:, dst, :].add(ys[:, src, :])


def kernel(x, edge_index, W1, b1, W2, b2):
    ei = edge_index.astype(jnp.int32)
    pad = EPAD - E
    src_m = jnp.concatenate(
        [ei[0], jnp.zeros((pad,), jnp.int32)]).reshape(ROWS, K)
    dst_m = jnp.concatenate(
        [ei[1], jnp.full((pad,), DUMMY, jnp.int32)]).reshape(ROWS, K)
    zeros = jnp.zeros((NPAD, DEGW), jnp.float32)
    ones = jnp.zeros((K, DEGW), jnp.float32).at[:, 0].set(1.0)

    deg_parts = _deg_kernel(dst_m, zeros, ones)
    ys = _prep(deg_parts, x)
    s1 = _prop_kernel(ys, src_m, dst_m)
    gs = _dense(deg_parts, s1, W1, b1.reshape(1, HID), W2)
    s2 = _prop_kernel(gs, src_m, dst_m)
    return _final(deg_parts, s2, b2.reshape(1, OUT_CH))


# speedup vs baseline: 10.6193x; 1.0009x over previous
import functools

import jax
import jax.numpy as jnp
from jax import lax
from jax.experimental import pallas as pl
from jax.experimental.pallas import tpu as pltpu
from jax.experimental.pallas import tpu_sc as plsc

N = 10000
IN_CH = 256
HID = 512
OUT_CH = 256
CH = 128
E = 160000
K = 128
ROWS = 1280
EPAD = ROWS * K
DUMMY = N
NPAD = 10240
NC, NS = 2, 16
RP = ROWS // NS
HK = RP // 2
RD = ROWS // (NC * NS)
NROWS_T = NPAD // NS
ZROWS_T = NPAD // NS
DEGW = 128
RB = 2000


def _mesh():
    return plsc.VectorSubcoreMesh(
        core_axis_name="c", subcore_axis_name="s",
        num_cores=NC, num_subcores=NS)


@functools.partial(
    pl.kernel,
    out_type=jax.ShapeDtypeStruct((NC, NPAD, DEGW), jnp.float32),
    mesh=_mesh(),
    scratch_types=[
        pltpu.VMEM((RD, K), jnp.int32),
        pltpu.VMEM((K, DEGW), jnp.float32),
        pltpu.VMEM_SHARED((NPAD, DEGW), jnp.float32),
    ],
)
def _deg_kernel(dst_hbm, zeros_hbm, ones_hbm, out_hbm, dst_v, ones_v, acc_sh):
    c = lax.axis_index("c")
    s = lax.axis_index("s")
    w = c * NS + s
    pltpu.sync_copy(zeros_hbm.at[pl.ds(s * ZROWS_T, ZROWS_T)],
                    acc_sh.at[pl.ds(s * ZROWS_T, ZROWS_T)])
    pltpu.sync_copy(dst_hbm.at[pl.ds(w * RD, RD)], dst_v)
    pltpu.sync_copy(ones_hbm, ones_v)
    plsc.subcore_barrier()

    def body(j, carry):
        pltpu.sync_copy(ones_v, acc_sh.at[dst_v.at[j]], add=True)
        return carry

    lax.fori_loop(0, RD, body, 0)
    plsc.subcore_barrier()
    pltpu.sync_copy(acc_sh.at[pl.ds(s * ZROWS_T, ZROWS_T)],
                    out_hbm.at[c, pl.ds(s * ZROWS_T, ZROWS_T)])


@functools.partial(
    pl.kernel,
    out_type=jax.ShapeDtypeStruct((NC, NPAD, CH), jnp.float32),
    mesh=_mesh(),
    scratch_types=[
        pltpu.VMEM((HK, K), jnp.int32),
        pltpu.VMEM((HK, K), jnp.int32),
        pltpu.VMEM((K, CH), jnp.float32),
        pltpu.VMEM((K, CH), jnp.float32),
        pltpu.VMEM_SHARED((NPAD, CH), jnp.float32),
        pltpu.SemaphoreType.DMA,
        pltpu.SemaphoreType.DMA,
    ],
)
def _prop_kernel(ys_hbm, src_hbm, dst_hbm, out_hbm,
                 src_v, dst_v, gb0, gb1, acc_sh, sem0, sem1):
    c = lax.axis_index("c")
    s = lax.axis_index("s")
    pltpu.sync_copy(ys_hbm.at[c, pl.ds(s * NROWS_T, NROWS_T)],
                    acc_sh.at[pl.ds(s * NROWS_T, NROWS_T)])
    plsc.subcore_barrier()

    gbufs = (gb0, gb1)
    sems = (sem0, sem1)

    def start(chunk, b):
        pltpu.async_copy(ys_hbm.at[c].at[src_v.at[chunk]], gbufs[b], sems[b])

    def wait(b):
        pltpu.make_async_copy(ys_hbm.at[c, pl.ds(0, K)],
                              gbufs[b], sems[b]).wait()

    def scat(chunk, b):
        pltpu.sync_copy(gbufs[b], acc_sh.at[dst_v.at[chunk]], add=True)

    def body(g, carry):
        ch0 = g * 2
        for b in range(2):
            wait(b)
            scat(ch0 + b, b)
            start(ch0 + b + 2, b)
        return carry

    for h in range(2):
        base = s * RP + h * HK
        pltpu.sync_copy(src_hbm.at[pl.ds(base, HK)], src_v)
        pltpu.sync_copy(dst_hbm.at[pl.ds(base, HK)], dst_v)
        start(0, 0)
        start(1, 1)
        lax.fori_loop(0, HK // 2 - 1, body, 0)
        for b in range(2):
            wait(b)
            scat(HK - 2 + b, b)
    plsc.subcore_barrier()
    pltpu.sync_copy(acc_sh.at[pl.ds(s * NROWS_T, NROWS_T)],
                    out_hbm.at[c, pl.ds(s * NROWS_T, NROWS_T)])


def _dis_of(p):
    return lax.rsqrt(1.0 + p[0, :, 0] + p[1, :, 0])


def _prep_body(deg_ref, x_ref, ys_ref):
    dis = _dis_of(deg_ref[...])
    xs = x_ref[...] * dis[:, None]
    ys_ref[0] = xs[:, :CH]
    ys_ref[1] = xs[:, CH:]


def _dense_body(deg_ref, s1_ref, w1_ref, b1_ref, w2_ref, out_ref):
    dis = _dis_of(deg_ref[...])
    s1 = jnp.concatenate([s1_ref[0], s1_ref[1]], axis=1) * dis[:, None]
    h = jnp.dot(s1, w1_ref[...], preferred_element_type=jnp.float32)
    h = jnp.maximum(h + b1_ref[...], 0.0)
    g = jnp.dot(h, w2_ref[...], preferred_element_type=jnp.float32)
    gs = g * dis[:, None]
    out_ref[0] = gs[:, :CH]
    out_ref[1] = gs[:, CH:]


def _out_body(deg_ref, s2_ref, b2_ref, out_ref):
    dis = _dis_of(deg_ref[...])
    p2 = jnp.concatenate([s2_ref[0], s2_ref[1]], axis=1) * dis[:, None]
    p2 = p2 + b2_ref[...]
    sh = p2 - jnp.max(p2, axis=1, keepdims=True)
    lse = jnp.log(jnp.sum(jnp.exp(sh), axis=1, keepdims=True))
    out_ref[...] = sh - lse


_DEG_SPEC = pl.BlockSpec((NC, RB, DEGW), lambda i: (0, i, 0))
_SPLIT_SPEC = pl.BlockSpec((NC, RB, CH), lambda i: (0, i, 0))


def _prep(deg_parts, x):
    return pl.pallas_call(
        _prep_body,
        grid=(N // RB,),
        in_specs=[_DEG_SPEC, pl.BlockSpec((RB, IN_CH), lambda i: (i, 0))],
        out_specs=_SPLIT_SPEC,
        out_shape=jax.ShapeDtypeStruct((NC, NPAD, CH), jnp.float32),
    )(deg_parts, x)


def _dense(deg_parts, s1, W1, b1, W2):
    return pl.pallas_call(
        _dense_body,
        grid=(N // RB,),
        in_specs=[
            _DEG_SPEC,
            _SPLIT_SPEC,
            pl.BlockSpec((IN_CH, HID), lambda i: (0, 0)),
            pl.BlockSpec((1, HID), lambda i: (0, 0)),
            pl.BlockSpec((HID, OUT_CH), lambda i: (0, 0)),
        ],
        out_specs=_SPLIT_SPEC,
        out_shape=jax.ShapeDtypeStruct((NC, NPAD, CH), jnp.float32),
    )(deg_parts, s1, W1, b1, W2)


def _final(deg_parts, s2, b2):
    return pl.pallas_call(
        _out_body,
        grid=(N // RB,),
        in_specs=[
            _DEG_SPEC,
            _SPLIT_SPEC,
            pl.BlockSpec((1, OUT_CH), lambda i: (0, 0)),
        ],
        out_specs=pl.BlockSpec((RB, OUT_CH), lambda i: (i, 0)),
        out_shape=jax.ShapeDtypeStruct((N, OUT_CH), jnp.float32),
    )(deg_parts, s2, b2)


def _prop_jnp(ys, src, dst):
    return ys.at[:, dst, :].add(ys[:, src, :])


def kernel(x, edge_index, W1, b1, W2, b2):
    ei = edge_index.astype(jnp.int32)
    pad = EPAD - E
    src_m = jnp.concatenate(
        [ei[0], jnp.zeros((pad,), jnp.int32)]).reshape(ROWS, K)
    dst_m = jnp.concatenate(
        [ei[1], jnp.full((pad,), DUMMY, jnp.int32)]).reshape(ROWS, K)
    zeros = jnp.zeros((NPAD, DEGW), jnp.float32)
    ones = jnp.zeros((K, DEGW), jnp.float32).at[:, 0].set(1.0)

    deg_parts = _deg_kernel(dst_m, zeros, ones)
    ys = _prep(deg_parts, x)
    s1 = _prop_kernel(ys, src_m, dst_m)
    gs = _dense(deg_parts, s1, W1, b1.reshape(1, HID), W2)
    s2 = _prop_kernel(gs, src_m, dst_m)
    return _final(deg_parts, s2, b2.reshape(1, OUT_CH))
